# baseline (speedup 1.0000x reference)
import functools

import jax
import jax.numpy as jnp
from jax import lax
from jax.experimental import pallas as pl
from jax.experimental.pallas import tpu as pltpu
from jax.experimental.pallas import tpu_sc as plsc

VOCAB = 1000000
SYN_NUM = 10
DPAD = 16
B = 4096
L = 200
N = B * L

NUM_CORES = 2
NUM_SUBCORES = 16
NUM_WORKERS = NUM_CORES * NUM_SUBCORES
B_PER_W = N // NUM_WORKERS
LANES = 16

VB = 2000
NB = VOCAB // VB
VB_VECS = VB // LANES

CHUNK = 1024
NUM_CHUNKS = B_PER_W // CHUNK
VECS = CHUNK // LANES


def _body(idx_hbm, synT_hbm, maskT_hbm, syns_out, mask_out,
          scr_s, scr_m, in_s, out_s, idx_c, syn_v, mask_v, syn_t, mask_t,
          sem_s, sem_m):
    ci = lax.axis_index("c")
    si = lax.axis_index("s")
    lane = lax.iota(jnp.int32, LANES)

    def relayout(tbl_hbm, scr):
        nblk = (NB - si + NUM_SUBCORES - 1) // NUM_SUBCORES

        def blk_step(bi, carry):
            g = si + bi * NUM_SUBCORES
            v0 = g * VB
            pltpu.sync_copy(tbl_hbm.at[:, pl.ds(v0, VB)], in_s)

            def t_step(i, carry2):
                rows = lane + i * LANES
                for c in range(SYN_NUM):
                    x = in_s[c, pl.ds(i * LANES, LANES)]
                    cols = jnp.full((LANES,), c, jnp.int32)
                    plsc.store_scatter(out_s, [rows, cols], x)
                return carry2

            lax.fori_loop(0, VB_VECS, t_step, 0)
            pltpu.sync_copy(out_s, scr.at[ci, pl.ds(v0, VB)])
            return carry

        lax.fori_loop(0, nblk, blk_step, 0)

    relayout(synT_hbm, scr_s)
    relayout(maskT_hbm, scr_m)
    plsc.subcore_barrier()

    wid = si * NUM_CORES + ci
    base_w = wid * B_PER_W

    def chunk_step(j, carry):
        base = base_w + j * CHUNK
        pltpu.sync_copy(idx_hbm.at[pl.ds(base, CHUNK)], idx_c)
        cp_s = pltpu.async_copy(scr_s.at[ci].at[idx_c], syn_v, sem_s)
        cp_m = pltpu.async_copy(scr_m.at[ci].at[idx_c], mask_v, sem_m)
        cp_s.wait()
        cp_m.wait()

        def vec_step(i, carry2):
            rows = lane + i * LANES
            for c in range(SYN_NUM):
                cols = jnp.full((LANES,), c, jnp.int32)
                sv = plsc.load_gather(syn_v, [rows, cols])
                syn_t[c, pl.ds(i * LANES, LANES)] = sv.astype(jnp.int32)
                mv = plsc.load_gather(mask_v, [rows, cols])
                mask_t[c, pl.ds(i * LANES, LANES)] = mv
            return carry2

        lax.fori_loop(0, VECS, vec_step, 0)
        for c in range(SYN_NUM):
            pltpu.sync_copy(syn_t.at[c], syns_out.at[c, pl.ds(base, CHUNK)])
            pltpu.sync_copy(mask_t.at[c], mask_out.at[c, pl.ds(base, CHUNK)])
        return carry

    lax.fori_loop(0, NUM_CHUNKS, chunk_step, 0)


_lookup = functools.partial(
    pl.kernel,
    out_type=(
        jax.ShapeDtypeStruct((SYN_NUM, N), jnp.int32),
        jax.ShapeDtypeStruct((SYN_NUM, N), jnp.float32),
    ),
    mesh=plsc.VectorSubcoreMesh(core_axis_name="c", subcore_axis_name="s"),
    scratch_types=[
        pltpu.HBM((NUM_CORES, VOCAB, DPAD), jnp.float32),
        pltpu.HBM((NUM_CORES, VOCAB, DPAD), jnp.float32),
        pltpu.VMEM((SYN_NUM, VB), jnp.float32),
        pltpu.VMEM((VB, DPAD), jnp.float32),
        pltpu.VMEM((CHUNK,), jnp.int32),
        pltpu.VMEM((CHUNK, DPAD), jnp.float32),
        pltpu.VMEM((CHUNK, DPAD), jnp.float32),
        pltpu.VMEM((SYN_NUM, CHUNK), jnp.int32),
        pltpu.VMEM((SYN_NUM, CHUNK), jnp.float32),
        pltpu.SemaphoreType.DMA,
        pltpu.SemaphoreType.DMA,
    ],
    compiler_params=pltpu.CompilerParams(
        use_tc_tiling_on_sc=False, needs_layout_passes=False),
)(_body)


@jax.jit
def kernel(idx, syn_table, mask_table):
    idx_t = idx.T.reshape(N)
    syns_pl, mask_pl = _lookup(idx_t, syn_table.T, mask_table.T)
    syns = jnp.transpose(syns_pl.reshape(SYN_NUM, L, B), (2, 1, 0))
    mask = jnp.transpose(mask_pl.reshape(SYN_NUM, L, B), (2, 1, 0))
    return syns, mask


# speedup vs baseline: 1.7386x; 1.7386x over previous
import functools

import jax
import jax.numpy as jnp
from jax import lax
from jax.experimental import pallas as pl
from jax.experimental.pallas import tpu as pltpu
from jax.experimental.pallas import tpu_sc as plsc

VOCAB = 1000000
SYN_NUM = 10
B = 4096
L = 200
N = B * L

VIEW_W = 16
VIEW_ROWS = VOCAB * SYN_NUM // VIEW_W

NUM_CORES = 2
NUM_SUBCORES = 16
NUM_WORKERS = NUM_CORES * NUM_SUBCORES
B_PER_W = N // NUM_WORKERS
CHUNK = 1024
NUM_CHUNKS = B_PER_W // CHUNK
LANES = 16
VECS = CHUNK // LANES


def _body(idx_hbm, syn_hbm, mask_hbm, syns_out, mask_out,
          idx_v, klo_v, khi_v, syn_b, mask_b, syn_t, mask_t, sem_s, sem_m):
    wid = lax.axis_index("s") * NUM_CORES + lax.axis_index("c")
    base_w = wid * B_PER_W
    pltpu.sync_copy(idx_hbm.at[pl.ds(base_w, B_PER_W)], idx_v)
    lane = lax.iota(jnp.int32, LANES)

    def chunk_step(j, carry):
        base = base_w + j * CHUNK

        def idx_step(i, carry2):
            v = idx_v[pl.ds(j * CHUNK + i * LANES, LANES)]
            klo = (v * SYN_NUM) >> 4
            klo_v[pl.ds(i * LANES, LANES)] = klo
            khi_v[pl.ds(i * LANES, LANES)] = jnp.minimum(klo + 1, VIEW_ROWS - 1)
            return carry2

        lax.fori_loop(0, VECS, idx_step, 0)
        cp_sl = pltpu.async_copy(syn_hbm.at[klo_v], syn_b.at[0], sem_s)
        cp_sh = pltpu.async_copy(syn_hbm.at[khi_v], syn_b.at[1], sem_s)
        cp_ml = pltpu.async_copy(mask_hbm.at[klo_v], mask_b.at[0], sem_m)
        cp_mh = pltpu.async_copy(mask_hbm.at[khi_v], mask_b.at[1], sem_m)
        cp_sl.wait()
        cp_sh.wait()
        cp_ml.wait()
        cp_mh.wait()

        def vec_step(i, carry2):
            rows = lane + i * LANES
            v = idx_v[pl.ds(j * CHUNK + i * LANES, LANES)]
            off = (v * SYN_NUM) & 15
            for c in range(SYN_NUM):
                t = off + c
                sel = t >> 4
                col = t & 15
                sv = plsc.load_gather(syn_b, [sel, rows, col])
                syn_t[c, pl.ds(i * LANES, LANES)] = sv.astype(jnp.int32)
                mv = plsc.load_gather(mask_b, [sel, rows, col])
                mask_t[c, pl.ds(i * LANES, LANES)] = mv
            return carry2

        lax.fori_loop(0, VECS, vec_step, 0)
        for c in range(SYN_NUM):
            pltpu.sync_copy(syn_t.at[c], syns_out.at[c, pl.ds(base, CHUNK)])
            pltpu.sync_copy(mask_t.at[c], mask_out.at[c, pl.ds(base, CHUNK)])
        return carry

    lax.fori_loop(0, NUM_CHUNKS, chunk_step, 0)


_lookup = functools.partial(
    pl.kernel,
    out_type=(
        jax.ShapeDtypeStruct((SYN_NUM, N), jnp.int32),
        jax.ShapeDtypeStruct((SYN_NUM, N), jnp.float32),
    ),
    mesh=plsc.VectorSubcoreMesh(core_axis_name="c", subcore_axis_name="s"),
    scratch_types=[
        pltpu.VMEM((B_PER_W,), jnp.int32),
        pltpu.VMEM((CHUNK,), jnp.int32),
        pltpu.VMEM((CHUNK,), jnp.int32),
        pltpu.VMEM((2, CHUNK, VIEW_W), jnp.float32),
        pltpu.VMEM((2, CHUNK, VIEW_W), jnp.float32),
        pltpu.VMEM((SYN_NUM, CHUNK), jnp.int32),
        pltpu.VMEM((SYN_NUM, CHUNK), jnp.float32),
        pltpu.SemaphoreType.DMA,
        pltpu.SemaphoreType.DMA,
    ],
    compiler_params=pltpu.CompilerParams(
        use_tc_tiling_on_sc=False, needs_layout_passes=False),
)(_body)


@jax.jit
def kernel(idx, syn_table, mask_table):
    idx_t = idx.T.reshape(N)
    syn_v16 = syn_table.reshape(VIEW_ROWS, VIEW_W)
    mask_v16 = mask_table.reshape(VIEW_ROWS, VIEW_W)
    syns_pl, mask_pl = _lookup(idx_t, syn_v16, mask_v16)
    syns = jnp.transpose(syns_pl.reshape(SYN_NUM, L, B), (2, 1, 0))
    mask = jnp.transpose(mask_pl.reshape(SYN_NUM, L, B), (2, 1, 0))
    return syns, mask


# speedup vs baseline: 1.9840x; 1.1412x over previous
import functools

import jax
import jax.numpy as jnp
from jax import lax
from jax.experimental import pallas as pl
from jax.experimental.pallas import tpu as pltpu
from jax.experimental.pallas import tpu_sc as plsc

VOCAB = 1000000
SYN_NUM = 10
B = 4096
L = 200
N = B * L

VIEW_W = 16
VIEW_ROWS = VOCAB * SYN_NUM // VIEW_W

NUM_CORES = 2
NUM_SUBCORES = 16
NUM_WORKERS = NUM_CORES * NUM_SUBCORES
B_PER_W = N // NUM_WORKERS
CHUNK = 1024
NUM_CHUNKS = B_PER_W // CHUNK
LANES = 16
VECS = CHUNK // LANES


def _make_body(out_dtype):
    convert = out_dtype == jnp.int32

    def _body(idx_hbm, tbl_hbm, out_hbm,
              idx_v, klo_v, khi_v, tbl_b, tbl_t, sem):
        wid = lax.axis_index("s") * NUM_CORES + lax.axis_index("c")
        base_w = wid * B_PER_W
        pltpu.sync_copy(idx_hbm.at[pl.ds(base_w, B_PER_W)], idx_v)
        lane = lax.iota(jnp.int32, LANES)

        def chunk_step(j, carry):
            base = base_w + j * CHUNK

            def idx_step(i, carry2):
                v = idx_v[pl.ds(j * CHUNK + i * LANES, LANES)]
                klo = (v * SYN_NUM) >> 4
                klo_v[pl.ds(i * LANES, LANES)] = klo
                khi_v[pl.ds(i * LANES, LANES)] = jnp.minimum(
                    klo + 1, VIEW_ROWS - 1)
                return carry2

            lax.fori_loop(0, VECS, idx_step, 0)
            cp_lo = pltpu.async_copy(tbl_hbm.at[klo_v], tbl_b.at[0], sem)
            cp_hi = pltpu.async_copy(tbl_hbm.at[khi_v], tbl_b.at[1], sem)
            cp_lo.wait()
            cp_hi.wait()

            def vec_step(i, carry2):
                rows = lane + i * LANES
                v = idx_v[pl.ds(j * CHUNK + i * LANES, LANES)]
                off = (v * SYN_NUM) & 15
                for c in range(SYN_NUM):
                    t = off + c
                    sel = t >> 4
                    col = t & 15
                    x = plsc.load_gather(tbl_b, [sel, rows, col])
                    if convert:
                        x = x.astype(jnp.int32)
                    tbl_t[c, pl.ds(i * LANES, LANES)] = x
                return carry2

            lax.fori_loop(0, VECS, vec_step, 0)
            for c in range(SYN_NUM):
                pltpu.sync_copy(tbl_t.at[c], out_hbm.at[c, pl.ds(base, CHUNK)])
            return carry

        lax.fori_loop(0, NUM_CHUNKS, chunk_step, 0)

    return _body


def _make_lookup(out_dtype):
    return functools.partial(
        pl.kernel,
        out_type=jax.ShapeDtypeStruct((SYN_NUM, N), out_dtype),
        mesh=plsc.VectorSubcoreMesh(core_axis_name="c", subcore_axis_name="s"),
        scratch_types=[
            pltpu.VMEM((B_PER_W,), jnp.int32),
            pltpu.VMEM((CHUNK,), jnp.int32),
            pltpu.VMEM((CHUNK,), jnp.int32),
            pltpu.VMEM((2, CHUNK, VIEW_W), jnp.float32),
            pltpu.VMEM((SYN_NUM, CHUNK), out_dtype),
            pltpu.SemaphoreType.DMA,
        ],
        compiler_params=pltpu.CompilerParams(
            use_tc_tiling_on_sc=False, needs_layout_passes=False),
    )(_make_body(out_dtype))


_lookup_syn = _make_lookup(jnp.int32)
_lookup_mask = _make_lookup(jnp.float32)


@jax.jit
def kernel(idx, syn_table, mask_table):
    idx_t = idx.T.reshape(N)
    syn_v16 = syn_table.reshape(VIEW_ROWS, VIEW_W)
    mask_v16 = mask_table.reshape(VIEW_ROWS, VIEW_W)
    syns_pl = _lookup_syn(idx_t, syn_v16)
    mask_pl = _lookup_mask(idx_t, mask_v16)
    syns = jnp.transpose(syns_pl.reshape(SYN_NUM, L, B), (2, 1, 0))
    mask = jnp.transpose(mask_pl.reshape(SYN_NUM, L, B), (2, 1, 0))
    return syns, mask


# speedup vs baseline: 2.1016x; 1.0593x over previous
import functools

import jax
import jax.numpy as jnp
from jax import lax
from jax.experimental import pallas as pl
from jax.experimental.pallas import tpu as pltpu
from jax.experimental.pallas import tpu_sc as plsc

VOCAB = 1000000
SYN_NUM = 10
B = 4096
L = 200
N = B * L

VIEW_W = 16
VIEW_ROWS = VOCAB * SYN_NUM // VIEW_W

NUM_CORES = 2
NUM_SUBCORES = 16
NUM_WORKERS = NUM_CORES * NUM_SUBCORES
B_PER_W = N // NUM_WORKERS
CHUNK = 1024
NUM_CHUNKS = B_PER_W // CHUNK
LANES = 16
VECS = CHUNK // LANES


def _make_body(out_dtype):
    convert = out_dtype == jnp.int32

    def _body(idx_hbm, tbl_hbm, out_hbm,
              idx_v, klo_v, khi_v, tbl_b, tbl_t, sem0, sem1):
        wid = lax.axis_index("s") * NUM_CORES + lax.axis_index("c")
        base_w = wid * B_PER_W
        pltpu.sync_copy(idx_hbm.at[pl.ds(base_w, B_PER_W)], idx_v)
        lane = lax.iota(jnp.int32, LANES)
        sems = (sem0, sem1)

        def issue(j, p):
            def idx_step(i, carry2):
                v = idx_v[pl.ds(j * CHUNK + i * LANES, LANES)]
                klo = (v * SYN_NUM) >> 4
                klo_v[p, pl.ds(i * LANES, LANES)] = klo
                khi_v[p, pl.ds(i * LANES, LANES)] = jnp.minimum(
                    klo + 1, VIEW_ROWS - 1)
                return carry2

            lax.fori_loop(0, VECS, idx_step, 0)
            cp_lo = pltpu.async_copy(
                tbl_hbm.at[klo_v.at[p]], tbl_b.at[p, 0], sems[p])
            cp_hi = pltpu.async_copy(
                tbl_hbm.at[khi_v.at[p]], tbl_b.at[p, 1], sems[p])
            return cp_lo, cp_hi

        def drain_extract(j, p, cps):
            base = base_w + j * CHUNK
            cps[0].wait()
            cps[1].wait()

            def vec_step(i, carry2):
                rows = lane + i * LANES
                v = idx_v[pl.ds(j * CHUNK + i * LANES, LANES)]
                off = (v * SYN_NUM) & 15
                for c in range(SYN_NUM):
                    t = off + c
                    sel = t >> 4
                    col = t & 15
                    x = plsc.load_gather(tbl_b.at[p], [sel, rows, col])
                    if convert:
                        x = x.astype(jnp.int32)
                    tbl_t[c, pl.ds(i * LANES, LANES)] = x
                return carry2

            lax.fori_loop(0, VECS, vec_step, 0)
            for c in range(SYN_NUM):
                pltpu.sync_copy(tbl_t.at[c], out_hbm.at[c, pl.ds(base, CHUNK)])

        cps = issue(0, 0)
        for j in range(NUM_CHUNKS - 1):
            nxt = issue(j + 1, (j + 1) % 2)
            drain_extract(j, j % 2, cps)
            cps = nxt
        drain_extract(NUM_CHUNKS - 1, (NUM_CHUNKS - 1) % 2, cps)

    return _body


def _make_lookup(out_dtype):
    return functools.partial(
        pl.kernel,
        out_type=jax.ShapeDtypeStruct((SYN_NUM, N), out_dtype),
        mesh=plsc.VectorSubcoreMesh(core_axis_name="c", subcore_axis_name="s"),
        scratch_types=[
            pltpu.VMEM((B_PER_W,), jnp.int32),
            pltpu.VMEM((2, CHUNK), jnp.int32),
            pltpu.VMEM((2, CHUNK), jnp.int32),
            pltpu.VMEM((2, 2, CHUNK, VIEW_W), jnp.float32),
            pltpu.VMEM((SYN_NUM, CHUNK), out_dtype),
            pltpu.SemaphoreType.DMA,
            pltpu.SemaphoreType.DMA,
        ],
        compiler_params=pltpu.CompilerParams(
            use_tc_tiling_on_sc=False, needs_layout_passes=False),
    )(_make_body(out_dtype))


_lookup_syn = _make_lookup(jnp.int32)
_lookup_mask = _make_lookup(jnp.float32)


@jax.jit
def kernel(idx, syn_table, mask_table):
    idx_t = idx.T.reshape(N)
    syn_v16 = syn_table.reshape(VIEW_ROWS, VIEW_W)
    mask_v16 = mask_table.reshape(VIEW_ROWS, VIEW_W)
    syns_pl = _lookup_syn(idx_t, syn_v16)
    mask_pl = _lookup_mask(idx_t, mask_v16)
    syns = jnp.transpose(syns_pl.reshape(SYN_NUM, L, B), (2, 1, 0))
    mask = jnp.transpose(mask_pl.reshape(SYN_NUM, L, B), (2, 1, 0))
    return syns, mask


# speedup vs baseline: 2.1036x; 1.0010x over previous
import functools

import jax
import jax.numpy as jnp
from jax import lax
from jax.experimental import pallas as pl
from jax.experimental.pallas import tpu as pltpu
from jax.experimental.pallas import tpu_sc as plsc

VOCAB = 1000000
SYN_NUM = 10
B = 4096
L = 200
N = B * L

VIEW_W = 16
VIEW_ROWS = VOCAB * SYN_NUM // VIEW_W

NUM_CORES = 2
NUM_SUBCORES = 16
NUM_WORKERS = NUM_CORES * NUM_SUBCORES
B_PER_W = N // NUM_WORKERS
CHUNK = 1280
NUM_CHUNKS = B_PER_W // CHUNK
LANES = 16
VECS = CHUNK // LANES


def _make_body(out_dtype):
    convert = out_dtype == jnp.int32

    def _body(idx_hbm, tbl_hbm, out_hbm,
              idx_v, klo_v, khi_v, tbl_b, tbl_t, sem0, sem1):
        wid = lax.axis_index("s") * NUM_CORES + lax.axis_index("c")
        base_w = wid * B_PER_W
        pltpu.sync_copy(idx_hbm.at[pl.ds(base_w, B_PER_W)], idx_v)
        lane = lax.iota(jnp.int32, LANES)
        sems = (sem0, sem1)

        def issue(j, p):
            def idx_step(i, carry2):
                v = idx_v[pl.ds(j * CHUNK + i * LANES, LANES)]
                klo = (v * SYN_NUM) >> 4
                klo_v[p, pl.ds(i * LANES, LANES)] = klo
                khi_v[p, pl.ds(i * LANES, LANES)] = jnp.minimum(
                    klo + 1, VIEW_ROWS - 1)
                return carry2

            lax.fori_loop(0, VECS, idx_step, 0)
            cp_lo = pltpu.async_copy(
                tbl_hbm.at[klo_v.at[p]], tbl_b.at[p, 0], sems[p])
            cp_hi = pltpu.async_copy(
                tbl_hbm.at[khi_v.at[p]], tbl_b.at[p, 1], sems[p])
            return cp_lo, cp_hi

        def drain_extract(j, p, cps):
            base = base_w + j * CHUNK
            cps[0].wait()
            cps[1].wait()

            def vec_step(i, carry2):
                rows = lane + i * LANES
                v = idx_v[pl.ds(j * CHUNK + i * LANES, LANES)]
                off = (v * SYN_NUM) & 15
                for c in range(SYN_NUM):
                    t = off + c
                    sel = t >> 4
                    col = t & 15
                    x = plsc.load_gather(tbl_b.at[p], [sel, rows, col])
                    if convert:
                        x = x.astype(jnp.int32)
                    tbl_t[c, pl.ds(i * LANES, LANES)] = x
                return carry2

            lax.fori_loop(0, VECS, vec_step, 0)
            for c in range(SYN_NUM):
                pltpu.sync_copy(tbl_t.at[c], out_hbm.at[c, pl.ds(base, CHUNK)])

        cps = issue(0, 0)
        for j in range(NUM_CHUNKS - 1):
            nxt = issue(j + 1, (j + 1) % 2)
            drain_extract(j, j % 2, cps)
            cps = nxt
        drain_extract(NUM_CHUNKS - 1, (NUM_CHUNKS - 1) % 2, cps)

    return _body


def _make_lookup(out_dtype):
    return functools.partial(
        pl.kernel,
        out_type=jax.ShapeDtypeStruct((SYN_NUM, N), out_dtype),
        mesh=plsc.VectorSubcoreMesh(core_axis_name="c", subcore_axis_name="s"),
        scratch_types=[
            pltpu.VMEM((B_PER_W,), jnp.int32),
            pltpu.VMEM((2, CHUNK), jnp.int32),
            pltpu.VMEM((2, CHUNK), jnp.int32),
            pltpu.VMEM((2, 2, CHUNK, VIEW_W), jnp.float32),
            pltpu.VMEM((SYN_NUM, CHUNK), out_dtype),
            pltpu.SemaphoreType.DMA,
            pltpu.SemaphoreType.DMA,
        ],
        compiler_params=pltpu.CompilerParams(
            use_tc_tiling_on_sc=False, needs_layout_passes=False),
    )(_make_body(out_dtype))


_lookup_syn = _make_lookup(jnp.int32)
_lookup_mask = _make_lookup(jnp.float32)


@jax.jit
def kernel(idx, syn_table, mask_table):
    idx_t = idx.T.reshape(N)
    syn_v16 = syn_table.reshape(VIEW_ROWS, VIEW_W)
    mask_v16 = mask_table.reshape(VIEW_ROWS, VIEW_W)
    syns_pl = _lookup_syn(idx_t, syn_v16)
    mask_pl = _lookup_mask(idx_t, mask_v16)
    syns = jnp.transpose(syns_pl.reshape(SYN_NUM, L, B), (2, 1, 0))
    mask = jnp.transpose(mask_pl.reshape(SYN_NUM, L, B), (2, 1, 0))
    return syns, mask
